# Initial kernel scaffold; baseline (speedup 1.0000x reference)
import jax
import jax.numpy as jnp
from jax.experimental import pallas as pl


def kernel(delta, v_old, G_idx):
    raise NotImplementedError("write your pallas kernel here")


# speedup vs baseline: 6.9015x; 6.9015x over previous
import functools

import jax
import jax.numpy as jnp
from jax import lax
from jax.experimental import pallas as pl
from jax.experimental.pallas import tpu as pltpu
from jax.experimental.pallas import tpu_sc as plsc

N = 1048576
D = 64
_NC = 2
_NS = 16
_NW = _NC * _NS
_L = 16
_C = 128
_RPW = N // _NW
_NCHUNK = _RPW // _C
_NPAIR = _NCHUNK // 2
_GPC = _C // _L

_mesh = plsc.VectorSubcoreMesh(core_axis_name="c", subcore_axis_name="s")


@functools.partial(
    pl.kernel,
    mesh=_mesh,
    out_type=jax.ShapeDtypeStruct((N, D), jnp.float32),
    scratch_types=[
        pltpu.VMEM((2, _C, D), jnp.float32),
        pltpu.VMEM((2, _C, D), jnp.float32),
        pltpu.VMEM((2, _C), jnp.int32),
        pltpu.VMEM((2, D), jnp.float32),
        pltpu.SemaphoreType.DMA,
        pltpu.SemaphoreType.DMA,
        pltpu.SemaphoreType.DMA,
        pltpu.SemaphoreType.DMA,
        pltpu.SemaphoreType.DMA,
        pltpu.SemaphoreType.DMA,
    ],
)
def _pv_kernel(delta_hbm, vold_hbm, g_hbm, out_hbm,
               inb, outb, gb, vb,
               sin_d0, sin_d1, sin_g0, sin_g1, sout0, sout1):
    sin_d = (sin_d0, sin_d1)
    sin_g = (sin_g0, sin_g1)
    sout = (sout0, sout1)
    wid = lax.axis_index("c") * _NS + lax.axis_index("s")
    wbase = wid * _RPW

    pltpu.sync_copy(vold_hbm, vb)
    dsub = [vb[0, pl.ds(_L * j, _L)] - vb[1, pl.ds(_L * j, _L)]
            for j in range(D // _L)]

    def in_copy_d(slot, i):
        return pltpu.make_async_copy(
            delta_hbm.at[pl.ds(wbase + i * _C, _C)], inb.at[slot], sin_d[slot])

    def in_copy_g(slot, i):
        return pltpu.make_async_copy(
            g_hbm.at[pl.ds(wbase + i * _C, _C)], gb.at[slot], sin_g[slot])

    def out_copy(slot, i):
        return pltpu.make_async_copy(
            outb.at[slot], out_hbm.at[pl.ds(wbase + i * _C, _C)], sout[slot])

    def compute_chunk(slot):
        def group(gidx, carry):
            row0 = gidx * _L
            gv = gb[slot, pl.ds(row0, _L)]
            fs = 1.0 - 2.0 * gv.astype(jnp.float32)
            for i in range(_L):
                s = fs.at[jnp.full((_L,), i, jnp.int32)].get(
                    mode="promise_in_bounds")
                for j in range(D // _L):
                    v = inb[slot, row0 + i, pl.ds(_L * j, _L)]
                    outb[slot, row0 + i, pl.ds(_L * j, _L)] = v * (s * dsub[j])
            return carry
        lax.fori_loop(0, _GPC, group, 0)

    in_copy_d(0, 0).start()
    in_copy_g(0, 0).start()
    in_copy_d(1, 1).start()
    in_copy_g(1, 1).start()

    def pair(p, carry):
        for slot in (0, 1):
            i = 2 * p + slot
            in_copy_d(slot, i).wait()
            in_copy_g(slot, i).wait()

            @pl.when(p > 0)
            def _wait_prev_out():
                out_copy(slot, i - 2).wait()

            compute_chunk(slot)
            out_copy(slot, i).start()

            @pl.when(p < _NPAIR - 1)
            def _start_next_in():
                in_copy_d(slot, i + 2).start()
                in_copy_g(slot, i + 2).start()
        return carry

    lax.fori_loop(0, _NPAIR, pair, 0)
    out_copy(0, _NCHUNK - 2).wait()
    out_copy(1, _NCHUNK - 1).wait()


def kernel(delta, v_old, G_idx):
    return _pv_kernel(delta, v_old, G_idx.astype(jnp.int32))
